# baseline (speedup 1.0000x reference)
import functools

import jax
import jax.numpy as jnp
from jax import lax
from jax.experimental import pallas as pl
from jax.experimental.pallas import tpu as pltpu
from jax.experimental.pallas import tpu_sc as plsc

C, H, W = 768, 224, 224
HW = H * W

_NC = 2
_NS = 16
_NW = _NC * _NS
_CPW = C // _NW

_CHUNK = 12544
_LANES = 16


def _shuffle_body(img_hbm, idx_hbm, out_hbm, idx_v, row_v, out_v):
    wid = lax.axis_index("s") * _NC + lax.axis_index("c")

    pltpu.sync_copy(idx_hbm, idx_v)

    def chan_body(i, carry):
        ch = wid * _CPW + i
        pltpu.sync_copy(img_hbm.at[ch], row_v)

        def chunk_body(k, carry2):
            def gather_body(j, carry3):
                base = k * _CHUNK + j * _LANES
                idx16 = idx_v[pl.ds(base, _LANES)]
                out_v[pl.ds(j * _LANES, _LANES)] = plsc.load_gather(
                    row_v, [idx16]
                )
                return carry3

            lax.fori_loop(0, _CHUNK // _LANES, gather_body, 0, unroll=4)
            pltpu.sync_copy(out_v, out_hbm.at[ch, pl.ds(k * _CHUNK, _CHUNK)])
            return carry2

        lax.fori_loop(0, HW // _CHUNK, chunk_body, 0)
        return carry

    lax.fori_loop(0, _CPW, chan_body, 0)


@jax.jit
def _shuffle(flat_img, idx32):
    mesh = plsc.VectorSubcoreMesh(core_axis_name="c", subcore_axis_name="s")
    fn = functools.partial(
        pl.kernel,
        mesh=mesh,
        compiler_params=pltpu.CompilerParams(needs_layout_passes=False),
        out_type=jax.ShapeDtypeStruct((C, HW), jnp.float32),
        scratch_types=[
            pltpu.VMEM((HW,), jnp.int32),
            pltpu.VMEM((HW,), jnp.float32),
            pltpu.VMEM((_CHUNK,), jnp.float32),
        ],
    )(_shuffle_body)
    return fn(flat_img, idx32)


def kernel(img, indices):
    Cc, Hh, Ww = img.shape
    flat = img.reshape(Cc, Hh * Ww)
    idx32 = indices.astype(jnp.int32)
    out = _shuffle(flat, idx32)
    return out.reshape(Cc, Hh, Ww)
